# baseline (speedup 1.0000x reference)
import functools

import jax
import jax.numpy as jnp
from jax import lax
from jax.experimental import pallas as pl
from jax.experimental.pallas import tpu as pltpu
from jax.experimental.pallas import tpu_sc as plsc

D = 128
L = 16
NW = 32
CH = 128
NSLOT = 3


def _tsvec_sc(batch):
  b_per_w = batch // NW
  n_chunks = b_per_w // CH
  mesh = plsc.VectorSubcoreMesh(core_axis_name="c", subcore_axis_name="s")

  @functools.partial(
      pl.kernel,
      mesh=mesh,
      compiler_params=pltpu.CompilerParams(needs_layout_passes=False),
      out_type=jax.ShapeDtypeStruct((batch,), jnp.float32),
      scratch_types=[
          pltpu.VMEM((n_chunks, CH), jnp.int32),
          pltpu.VMEM((n_chunks, CH), jnp.int32),
          pltpu.VMEM((n_chunks, CH), jnp.int32),
          pltpu.VMEM((NSLOT, CH, D), jnp.float32),
          pltpu.VMEM((NSLOT, CH, D), jnp.float32),
          pltpu.VMEM((b_per_w * L,), jnp.float32),
          pltpu.VMEM((b_per_w,), jnp.float32),
          pltpu.SemaphoreType.DMA((NSLOT,)),
          pltpu.SemaphoreType.DMA((NSLOT,)),
          pltpu.SemaphoreType.DMA((NSLOT,)),
      ],
  )
  def k(head_hbm, rel_hbm, tail_hbm, ent_hbm, relemb_hbm, out_hbm,
        hidx, ridx, tidx, hbuf, tbuf, part, obuf, hsem, tsem, rsem):
    wid = lax.axis_index("s") * 2 + lax.axis_index("c")
    base_row = wid * b_per_w

    for ci in range(n_chunks):
      span = pl.ds(base_row + ci * CH, CH)
      pltpu.sync_copy(head_hbm.at[span], hidx.at[ci])
      pltpu.sync_copy(rel_hbm.at[span], ridx.at[ci])
      pltpu.sync_copy(tail_hbm.at[span], tidx.at[ci])

    def fire_ht(ci):
      slot = ci % NSLOT
      pltpu.async_copy(ent_hbm.at[hidx.at[ci]], hbuf.at[slot], hsem.at[slot])
      pltpu.async_copy(ent_hbm.at[tidx.at[ci]], tbuf.at[slot], tsem.at[slot])

    def wait_h(ci):
      slot = ci % NSLOT
      pltpu.make_async_copy(ent_hbm.at[hidx.at[ci]], hbuf.at[slot],
                            hsem.at[slot]).wait()

    def fire_radd(ci):
      slot = ci % NSLOT
      pltpu.async_copy(relemb_hbm.at[ridx.at[ci]], hbuf.at[slot],
                       rsem.at[slot], add=True)

    def wait_tr(ci):
      slot = ci % NSLOT
      pltpu.make_async_copy(ent_hbm.at[tidx.at[ci]], tbuf.at[slot],
                            tsem.at[slot]).wait()
      pltpu.make_async_copy(relemb_hbm.at[ridx.at[ci]], hbuf.at[slot],
                            rsem.at[slot]).wait()

    fire_ht(0)
    if n_chunks > 1:
      fire_ht(1)
    wait_h(0)
    fire_radd(0)

    for ci in range(n_chunks):
      slot = ci % NSLOT
      if ci + 2 < n_chunks:
        fire_ht(ci + 2)
      if ci + 1 < n_chunks:
        wait_h(ci + 1)
        fire_radd(ci + 1)
      wait_tr(ci)

      U = 8

      @plsc.parallel_loop(0, CH // U)
      def row_body(it):
        base = it * U
        for j in range(U):
          acc = jnp.zeros((L,), jnp.float32)
          for c in range(D // L):
            hv = hbuf[slot, base + j, pl.ds(c * L, L)]
            tv = tbuf[slot, base + j, pl.ds(c * L, L)]
            d = hv - tv
            acc = acc + d * d
          part[pl.ds((ci * CH + base + j) * L, L)] = acc

    lanes = lax.iota(jnp.int32, L)
    for blk in range(b_per_w // L):
      x = jnp.zeros((L,), jnp.float32)
      for j in range(L):
        pv = part[pl.ds((blk * L + j) * L, L)]
        x = jnp.where(lanes == j, jnp.sum(pv), x)
      i = plsc.bitcast(x, jnp.int32)
      i = jnp.int32(0x5F3759DF) - (i >> 1)
      y = plsc.bitcast(i, jnp.float32)
      half_x = 0.5 * x
      for _ in range(3):
        y = y * (1.5 - half_x * y * y)
      obuf[pl.ds(blk * L, L)] = -(x * y)

    pltpu.sync_copy(obuf, out_hbm.at[pl.ds(base_row, b_per_w)])

  return k


def kernel(head, relation, tail, entity_emb, relation_emb):
  batch = head.shape[0]
  return _tsvec_sc(batch)(head, relation, tail, entity_emb, relation_emb)


# speedup vs baseline: 1.1190x; 1.1190x over previous
import functools

import jax
import jax.numpy as jnp
from jax import lax
from jax.experimental import pallas as pl
from jax.experimental.pallas import tpu as pltpu
from jax.experimental.pallas import tpu_sc as plsc

D = 128
L = 16
NW = 32
CH = 128
NSLOT = 3


def _tsvec_sc(batch):
  b_per_w = batch // NW
  n_chunks = b_per_w // CH
  mesh = plsc.VectorSubcoreMesh(core_axis_name="c", subcore_axis_name="s")

  @functools.partial(
      pl.kernel,
      mesh=mesh,
      compiler_params=pltpu.CompilerParams(needs_layout_passes=False),
      out_type=jax.ShapeDtypeStruct((batch,), jnp.float32),
      scratch_types=[
          pltpu.VMEM((n_chunks, CH), jnp.int32),
          pltpu.VMEM((n_chunks, CH), jnp.int32),
          pltpu.VMEM((n_chunks, CH), jnp.int32),
          pltpu.VMEM((NSLOT, CH, D), jnp.float32),
          pltpu.VMEM((NSLOT, CH, D), jnp.float32),
          pltpu.VMEM((b_per_w * L,), jnp.float32),
          pltpu.VMEM((b_per_w,), jnp.float32),
          pltpu.SemaphoreType.DMA((NSLOT,)),
          pltpu.SemaphoreType.DMA((NSLOT,)),
          pltpu.SemaphoreType.DMA((NSLOT,)),
          pltpu.SemaphoreType.DMA,
      ],
  )
  def k(head_hbm, rel_hbm, tail_hbm, ent_hbm, relemb_hbm, out_hbm,
        hidx, ridx, tidx, hbuf, tbuf, part, obuf, hsem, tsem, rsem, isem):
    wid = lax.axis_index("s") * 2 + lax.axis_index("c")
    base_row = wid * b_per_w

    idx_copies = []
    for ci in range(n_chunks):
      span = pl.ds(base_row + ci * CH, CH)
      idx_copies.append(pltpu.async_copy(head_hbm.at[span], hidx.at[ci], isem))
      idx_copies.append(pltpu.async_copy(rel_hbm.at[span], ridx.at[ci], isem))
      idx_copies.append(pltpu.async_copy(tail_hbm.at[span], tidx.at[ci], isem))
    for c in idx_copies:
      c.wait()

    def fire_ht(ci):
      slot = ci % NSLOT
      pltpu.async_copy(ent_hbm.at[hidx.at[ci]], hbuf.at[slot], hsem.at[slot])
      pltpu.async_copy(ent_hbm.at[tidx.at[ci]], tbuf.at[slot], tsem.at[slot])

    def wait_h(ci):
      slot = ci % NSLOT
      pltpu.make_async_copy(ent_hbm.at[hidx.at[ci]], hbuf.at[slot],
                            hsem.at[slot]).wait()

    def fire_radd(ci):
      slot = ci % NSLOT
      pltpu.async_copy(relemb_hbm.at[ridx.at[ci]], hbuf.at[slot],
                       rsem.at[slot], add=True)

    def wait_tr(ci):
      slot = ci % NSLOT
      pltpu.make_async_copy(ent_hbm.at[tidx.at[ci]], tbuf.at[slot],
                            tsem.at[slot]).wait()
      pltpu.make_async_copy(relemb_hbm.at[ridx.at[ci]], hbuf.at[slot],
                            rsem.at[slot]).wait()

    fire_ht(0)
    if n_chunks > 1:
      fire_ht(1)
    wait_h(0)
    fire_radd(0)

    for ci in range(n_chunks):
      slot = ci % NSLOT
      if ci + 2 < n_chunks:
        fire_ht(ci + 2)
      if ci + 1 < n_chunks:
        wait_h(ci + 1)
        fire_radd(ci + 1)
      wait_tr(ci)

      U = 8

      @plsc.parallel_loop(0, CH // U)
      def row_body(it):
        base = it * U
        for j in range(U):
          acc = jnp.zeros((L,), jnp.float32)
          for c in range(D // L):
            hv = hbuf[slot, base + j, pl.ds(c * L, L)]
            tv = tbuf[slot, base + j, pl.ds(c * L, L)]
            d = hv - tv
            acc = acc + d * d
          part[pl.ds((ci * CH + base + j) * L, L)] = acc

    lanes = lax.iota(jnp.int32, L)
    for blk in range(b_per_w // L):
      x = jnp.zeros((L,), jnp.float32)
      for j in range(L):
        pv = part[pl.ds((blk * L + j) * L, L)]
        x = jnp.where(lanes == j, jnp.sum(pv), x)
      i = plsc.bitcast(x, jnp.int32)
      i = jnp.int32(0x5F3759DF) - (i >> 1)
      y = plsc.bitcast(i, jnp.float32)
      half_x = 0.5 * x
      for _ in range(3):
        y = y * (1.5 - half_x * y * y)
      obuf[pl.ds(blk * L, L)] = -(x * y)

    pltpu.sync_copy(obuf, out_hbm.at[pl.ds(base_row, b_per_w)])

  return k


def kernel(head, relation, tail, entity_emb, relation_emb):
  batch = head.shape[0]
  return _tsvec_sc(batch)(head, relation, tail, entity_emb, relation_emb)


# speedup vs baseline: 1.1990x; 1.0715x over previous
import functools

import jax
import jax.numpy as jnp
from jax import lax
from jax.experimental import pallas as pl
from jax.experimental.pallas import tpu as pltpu
from jax.experimental.pallas import tpu_sc as plsc

D = 128
L = 16
NW = 32
CH = 128
NSLOT = 3


def _tsvec_sc(batch):
  b_per_w = batch // NW
  n_chunks = b_per_w // CH
  mesh = plsc.VectorSubcoreMesh(core_axis_name="c", subcore_axis_name="s")

  @functools.partial(
      pl.kernel,
      mesh=mesh,
      compiler_params=pltpu.CompilerParams(needs_layout_passes=False),
      out_type=jax.ShapeDtypeStruct((batch,), jnp.float32),
      scratch_types=[
          pltpu.VMEM((n_chunks, CH), jnp.int32),
          pltpu.VMEM((n_chunks, CH), jnp.int32),
          pltpu.VMEM((n_chunks, CH), jnp.int32),
          pltpu.VMEM((NSLOT, CH, D), jnp.float32),
          pltpu.VMEM((NSLOT, CH, D), jnp.float32),
          pltpu.VMEM((b_per_w * L,), jnp.float32),
          pltpu.VMEM((b_per_w,), jnp.float32),
          pltpu.SemaphoreType.DMA((NSLOT,)),
          pltpu.SemaphoreType.DMA((NSLOT,)),
          pltpu.SemaphoreType.DMA((NSLOT,)),
          pltpu.SemaphoreType.DMA,
      ],
  )
  def k(head_hbm, rel_hbm, tail_hbm, ent_hbm, relemb_hbm, out_hbm,
        hidx, ridx, tidx, hbuf, tbuf, part, obuf, hsem, tsem, rsem, isem):
    wid = lax.axis_index("s") * 2 + lax.axis_index("c")
    base_row = wid * b_per_w

    idx_copies = []
    for ci in range(n_chunks):
      span = pl.ds(base_row + ci * CH, CH)
      idx_copies.append(pltpu.async_copy(head_hbm.at[span], hidx.at[ci], isem))
      idx_copies.append(pltpu.async_copy(rel_hbm.at[span], ridx.at[ci], isem))
      idx_copies.append(pltpu.async_copy(tail_hbm.at[span], tidx.at[ci], isem))
    for c in idx_copies:
      c.wait()

    def fire_ht(ci):
      slot = ci % NSLOT
      pltpu.async_copy(ent_hbm.at[hidx.at[ci]], hbuf.at[slot], hsem.at[slot])
      pltpu.async_copy(ent_hbm.at[tidx.at[ci]], tbuf.at[slot], tsem.at[slot])

    def wait_h(ci):
      slot = ci % NSLOT
      pltpu.make_async_copy(ent_hbm.at[hidx.at[ci]], hbuf.at[slot],
                            hsem.at[slot]).wait()

    def fire_radd(ci):
      slot = ci % NSLOT
      pltpu.async_copy(relemb_hbm.at[ridx.at[ci]], hbuf.at[slot],
                       rsem.at[slot], add=True)

    def wait_tr(ci):
      slot = ci % NSLOT
      pltpu.make_async_copy(ent_hbm.at[tidx.at[ci]], tbuf.at[slot],
                            tsem.at[slot]).wait()
      pltpu.make_async_copy(relemb_hbm.at[ridx.at[ci]], hbuf.at[slot],
                            rsem.at[slot]).wait()

    fire_ht(0)
    if n_chunks > 1:
      fire_ht(1)
    wait_h(0)
    fire_radd(0)

    for ci in range(n_chunks):
      slot = ci % NSLOT
      if ci + 2 < n_chunks:
        fire_ht(ci + 2)
      if ci + 1 < n_chunks:
        wait_h(ci + 1)
        fire_radd(ci + 1)
      wait_tr(ci)

      U = 4

      @plsc.parallel_loop(0, CH // U)
      def row_body(it):
        base = it * U
        for j in range(U):
          acc = jnp.zeros((L,), jnp.float32)
          for c in range(D // L):
            hv = hbuf[slot, base + j, pl.ds(c * L, L)]
            tv = tbuf[slot, base + j, pl.ds(c * L, L)]
            d = hv - tv
            acc = acc + d * d
          part[pl.ds((ci * CH + base + j) * L, L)] = acc

    lanes = lax.iota(jnp.int32, L)
    for blk in range(b_per_w // L):
      x = jnp.zeros((L,), jnp.float32)
      for j in range(L):
        pv = part[pl.ds((blk * L + j) * L, L)]
        x = jnp.where(lanes == j, jnp.sum(pv), x)
      i = plsc.bitcast(x, jnp.int32)
      i = jnp.int32(0x5F3759DF) - (i >> 1)
      y = plsc.bitcast(i, jnp.float32)
      half_x = 0.5 * x
      for _ in range(3):
        y = y * (1.5 - half_x * y * y)
      obuf[pl.ds(blk * L, L)] = -(x * y)

    pltpu.sync_copy(obuf, out_hbm.at[pl.ds(base_row, b_per_w)])

  return k


def kernel(head, relation, tail, entity_emb, relation_emb):
  batch = head.shape[0]
  return _tsvec_sc(batch)(head, relation, tail, entity_emb, relation_emb)


# speedup vs baseline: 1.2426x; 1.0364x over previous
import functools

import jax
import jax.numpy as jnp
from jax import lax
from jax.experimental import pallas as pl
from jax.experimental.pallas import tpu as pltpu
from jax.experimental.pallas import tpu_sc as plsc

D = 128
L = 16
NW = 32
CH = 128
NSLOT = 3


def _tsvec_sc(batch):
  b_per_w = batch // NW
  n_chunks = b_per_w // CH
  mesh = plsc.VectorSubcoreMesh(core_axis_name="c", subcore_axis_name="s")

  @functools.partial(
      pl.kernel,
      mesh=mesh,
      compiler_params=pltpu.CompilerParams(needs_layout_passes=False),
      out_type=jax.ShapeDtypeStruct((batch,), jnp.float32),
      scratch_types=[
          pltpu.VMEM((n_chunks, CH), jnp.int32),
          pltpu.VMEM((n_chunks, CH), jnp.int32),
          pltpu.VMEM((n_chunks, CH), jnp.int32),
          pltpu.VMEM((NSLOT, CH, D), jnp.float32),
          pltpu.VMEM((NSLOT, CH, D), jnp.float32),
          pltpu.VMEM((b_per_w * L,), jnp.float32),
          pltpu.VMEM((b_per_w,), jnp.float32),
          pltpu.SemaphoreType.DMA((NSLOT,)),
          pltpu.SemaphoreType.DMA((NSLOT,)),
          pltpu.SemaphoreType.DMA((NSLOT,)),
          pltpu.SemaphoreType.DMA,
      ],
  )
  def k(head_hbm, rel_hbm, tail_hbm, ent_hbm, relemb_hbm, out_hbm,
        hidx, ridx, tidx, hbuf, tbuf, part, obuf, hsem, tsem, rsem, isem):
    wid = lax.axis_index("s") * 2 + lax.axis_index("c")
    base_row = wid * b_per_w

    idx_copies = []
    for ci in range(n_chunks):
      span = pl.ds(base_row + ci * CH, CH)
      idx_copies.append(pltpu.async_copy(head_hbm.at[span], hidx.at[ci], isem))
      idx_copies.append(pltpu.async_copy(rel_hbm.at[span], ridx.at[ci], isem))
      idx_copies.append(pltpu.async_copy(tail_hbm.at[span], tidx.at[ci], isem))
    for c in idx_copies:
      c.wait()

    def fire_ht(ci):
      slot = ci % NSLOT
      pltpu.async_copy(ent_hbm.at[hidx.at[ci]], hbuf.at[slot], hsem.at[slot])
      pltpu.async_copy(ent_hbm.at[tidx.at[ci]], tbuf.at[slot], tsem.at[slot])

    def wait_h(ci):
      slot = ci % NSLOT
      pltpu.make_async_copy(ent_hbm.at[hidx.at[ci]], hbuf.at[slot],
                            hsem.at[slot]).wait()

    def fire_radd(ci):
      slot = ci % NSLOT
      pltpu.async_copy(relemb_hbm.at[ridx.at[ci]], hbuf.at[slot],
                       rsem.at[slot], add=True)

    def wait_tr(ci):
      slot = ci % NSLOT
      pltpu.make_async_copy(ent_hbm.at[tidx.at[ci]], tbuf.at[slot],
                            tsem.at[slot]).wait()
      pltpu.make_async_copy(relemb_hbm.at[ridx.at[ci]], hbuf.at[slot],
                            rsem.at[slot]).wait()

    fire_ht(0)
    if n_chunks > 1:
      fire_ht(1)
    wait_h(0)
    fire_radd(0)

    for ci in range(n_chunks):
      slot = ci % NSLOT
      if ci + 2 < n_chunks:
        fire_ht(ci + 2)
      if ci + 1 < n_chunks:
        wait_h(ci + 1)
        fire_radd(ci + 1)
      wait_tr(ci)

      U = 2

      @plsc.parallel_loop(0, CH // U)
      def row_body(it):
        base = it * U
        for j in range(U):
          acc = jnp.zeros((L,), jnp.float32)
          for c in range(D // L):
            hv = hbuf[slot, base + j, pl.ds(c * L, L)]
            tv = tbuf[slot, base + j, pl.ds(c * L, L)]
            d = hv - tv
            acc = acc + d * d
          part[pl.ds((ci * CH + base + j) * L, L)] = acc

    lanes = lax.iota(jnp.int32, L)
    for blk in range(b_per_w // L):
      x = jnp.zeros((L,), jnp.float32)
      for j in range(L):
        pv = part[pl.ds((blk * L + j) * L, L)]
        x = jnp.where(lanes == j, jnp.sum(pv), x)
      i = plsc.bitcast(x, jnp.int32)
      i = jnp.int32(0x5F3759DF) - (i >> 1)
      y = plsc.bitcast(i, jnp.float32)
      half_x = 0.5 * x
      for _ in range(3):
        y = y * (1.5 - half_x * y * y)
      obuf[pl.ds(blk * L, L)] = -(x * y)

    pltpu.sync_copy(obuf, out_hbm.at[pl.ds(base_row, b_per_w)])

  return k


def kernel(head, relation, tail, entity_emb, relation_emb):
  batch = head.shape[0]
  return _tsvec_sc(batch)(head, relation, tail, entity_emb, relation_emb)


# speedup vs baseline: 1.2580x; 1.0124x over previous
import functools

import jax
import jax.numpy as jnp
from jax import lax
from jax.experimental import pallas as pl
from jax.experimental.pallas import tpu as pltpu
from jax.experimental.pallas import tpu_sc as plsc

D = 128
L = 16
NW = 32
CH = 128
NSLOT = 3


def _tsvec_sc(batch):
  b_per_w = batch // NW
  n_chunks = b_per_w // CH
  mesh = plsc.VectorSubcoreMesh(core_axis_name="c", subcore_axis_name="s")

  @functools.partial(
      pl.kernel,
      mesh=mesh,
      compiler_params=pltpu.CompilerParams(needs_layout_passes=False),
      out_type=jax.ShapeDtypeStruct((batch,), jnp.float32),
      scratch_types=[
          pltpu.VMEM((n_chunks, CH), jnp.int32),
          pltpu.VMEM((n_chunks, CH), jnp.int32),
          pltpu.VMEM((n_chunks, CH), jnp.int32),
          pltpu.VMEM((NSLOT, CH, D), jnp.float32),
          pltpu.VMEM((NSLOT, CH, D), jnp.float32),
          pltpu.VMEM((b_per_w * L,), jnp.float32),
          pltpu.VMEM((b_per_w,), jnp.float32),
          pltpu.SemaphoreType.DMA((NSLOT,)),
          pltpu.SemaphoreType.DMA((NSLOT,)),
          pltpu.SemaphoreType.DMA((NSLOT,)),
          pltpu.SemaphoreType.DMA,
      ],
  )
  def k(head_hbm, rel_hbm, tail_hbm, ent_hbm, relemb_hbm, out_hbm,
        hidx, ridx, tidx, hbuf, tbuf, part, obuf, hsem, tsem, rsem, isem):
    wid = lax.axis_index("s") * 2 + lax.axis_index("c")
    base_row = wid * b_per_w

    idx_copies = []
    for ci in range(n_chunks):
      span = pl.ds(base_row + ci * CH, CH)
      idx_copies.append(pltpu.async_copy(head_hbm.at[span], hidx.at[ci], isem))
      idx_copies.append(pltpu.async_copy(rel_hbm.at[span], ridx.at[ci], isem))
      idx_copies.append(pltpu.async_copy(tail_hbm.at[span], tidx.at[ci], isem))
    for c in idx_copies:
      c.wait()

    def fire_ht(ci):
      slot = ci % NSLOT
      pltpu.async_copy(ent_hbm.at[hidx.at[ci]], hbuf.at[slot], hsem.at[slot])
      pltpu.async_copy(ent_hbm.at[tidx.at[ci]], tbuf.at[slot], tsem.at[slot])

    def wait_h(ci):
      slot = ci % NSLOT
      pltpu.make_async_copy(ent_hbm.at[hidx.at[ci]], hbuf.at[slot],
                            hsem.at[slot]).wait()

    def fire_radd(ci):
      slot = ci % NSLOT
      pltpu.async_copy(relemb_hbm.at[ridx.at[ci]], hbuf.at[slot],
                       rsem.at[slot], add=True)

    def wait_tr(ci):
      slot = ci % NSLOT
      pltpu.make_async_copy(ent_hbm.at[tidx.at[ci]], tbuf.at[slot],
                            tsem.at[slot]).wait()
      pltpu.make_async_copy(relemb_hbm.at[ridx.at[ci]], hbuf.at[slot],
                            rsem.at[slot]).wait()

    fire_ht(0)
    if n_chunks > 1:
      fire_ht(1)
    wait_h(0)
    fire_radd(0)

    for ci in range(n_chunks):
      slot = ci % NSLOT
      if ci + 2 < n_chunks:
        fire_ht(ci + 2)
      if ci + 1 < n_chunks:
        wait_h(ci + 1)
        fire_radd(ci + 1)
      wait_tr(ci)

      U = 1

      @plsc.parallel_loop(0, CH // U)
      def row_body(it):
        base = it * U
        for j in range(U):
          acc = jnp.zeros((L,), jnp.float32)
          for c in range(D // L):
            hv = hbuf[slot, base + j, pl.ds(c * L, L)]
            tv = tbuf[slot, base + j, pl.ds(c * L, L)]
            d = hv - tv
            acc = acc + d * d
          part[pl.ds((ci * CH + base + j) * L, L)] = acc

    lanes = lax.iota(jnp.int32, L)
    for blk in range(b_per_w // L):
      x = jnp.zeros((L,), jnp.float32)
      for j in range(L):
        pv = part[pl.ds((blk * L + j) * L, L)]
        x = jnp.where(lanes == j, jnp.sum(pv), x)
      i = plsc.bitcast(x, jnp.int32)
      i = jnp.int32(0x5F3759DF) - (i >> 1)
      y = plsc.bitcast(i, jnp.float32)
      half_x = 0.5 * x
      for _ in range(3):
        y = y * (1.5 - half_x * y * y)
      obuf[pl.ds(blk * L, L)] = -(x * y)

    pltpu.sync_copy(obuf, out_hbm.at[pl.ds(base_row, b_per_w)])

  return k


def kernel(head, relation, tail, entity_emb, relation_emb):
  batch = head.shape[0]
  return _tsvec_sc(batch)(head, relation, tail, entity_emb, relation_emb)


# speedup vs baseline: 1.2768x; 1.0149x over previous
import functools

import jax
import jax.numpy as jnp
from jax import lax
from jax.experimental import pallas as pl
from jax.experimental.pallas import tpu as pltpu
from jax.experimental.pallas import tpu_sc as plsc

D = 128
L = 16
NW = 32
CH = 128
NSLOT = 3


def _tsvec_sc(batch):
  b_per_w = batch // NW
  n_chunks = b_per_w // CH
  mesh = plsc.VectorSubcoreMesh(core_axis_name="c", subcore_axis_name="s")

  @functools.partial(
      pl.kernel,
      mesh=mesh,
      compiler_params=pltpu.CompilerParams(needs_layout_passes=False),
      out_type=jax.ShapeDtypeStruct((batch,), jnp.float32),
      scratch_types=[
          pltpu.VMEM((n_chunks, CH), jnp.int32),
          pltpu.VMEM((n_chunks, CH), jnp.int32),
          pltpu.VMEM((n_chunks, CH), jnp.int32),
          pltpu.VMEM((NSLOT, CH, D), jnp.float32),
          pltpu.VMEM((NSLOT, CH, D), jnp.float32),
          pltpu.VMEM((b_per_w * L,), jnp.float32),
          pltpu.VMEM((b_per_w,), jnp.float32),
          pltpu.SemaphoreType.DMA((NSLOT,)),
          pltpu.SemaphoreType.DMA((NSLOT,)),
          pltpu.SemaphoreType.DMA((NSLOT,)),
          pltpu.SemaphoreType.DMA,
      ],
  )
  def k(head_hbm, rel_hbm, tail_hbm, ent_hbm, relemb_hbm, out_hbm,
        hidx, ridx, tidx, hbuf, tbuf, part, obuf, hsem, tsem, rsem, isem):
    wid = lax.axis_index("s") * 2 + lax.axis_index("c")
    base_row = wid * b_per_w

    idx_copies = []
    for ci in range(n_chunks):
      span = pl.ds(base_row + ci * CH, CH)
      idx_copies.append(pltpu.async_copy(head_hbm.at[span], hidx.at[ci], isem))
      idx_copies.append(pltpu.async_copy(rel_hbm.at[span], ridx.at[ci], isem))
      idx_copies.append(pltpu.async_copy(tail_hbm.at[span], tidx.at[ci], isem))
    for c in idx_copies:
      c.wait()

    def fire_ht(ci):
      slot = ci % NSLOT
      pltpu.async_copy(ent_hbm.at[hidx.at[ci]], hbuf.at[slot], hsem.at[slot])
      pltpu.async_copy(ent_hbm.at[tidx.at[ci]], tbuf.at[slot], tsem.at[slot])

    def wait_h(ci):
      slot = ci % NSLOT
      pltpu.make_async_copy(ent_hbm.at[hidx.at[ci]], hbuf.at[slot],
                            hsem.at[slot]).wait()

    def fire_radd(ci):
      slot = ci % NSLOT
      pltpu.async_copy(relemb_hbm.at[ridx.at[ci]], hbuf.at[slot],
                       rsem.at[slot], add=True)

    def wait_tr(ci):
      slot = ci % NSLOT
      pltpu.make_async_copy(ent_hbm.at[tidx.at[ci]], tbuf.at[slot],
                            tsem.at[slot]).wait()
      pltpu.make_async_copy(relemb_hbm.at[ridx.at[ci]], hbuf.at[slot],
                            rsem.at[slot]).wait()

    fire_ht(0)
    if n_chunks > 1:
      fire_ht(1)
    wait_h(0)
    fire_radd(0)

    for ci in range(n_chunks):
      slot = ci % NSLOT
      if ci + 2 < n_chunks:
        fire_ht(ci + 2)
      if ci + 1 < n_chunks:
        wait_h(ci + 1)
        fire_radd(ci + 1)
      wait_tr(ci)

      U = 1

      @plsc.parallel_loop(0, CH // U)
      def row_body(it):
        base = it * U
        for j in range(U):
          acc = jnp.zeros((L,), jnp.float32)
          for c in range(D // L):
            hv = hbuf[slot, base + j, pl.ds(c * L, L)]
            tv = tbuf[slot, base + j, pl.ds(c * L, L)]
            d = hv - tv
            acc = acc + d * d
          part[pl.ds((ci * CH + base + j) * L, L)] = acc

    lanes = lax.iota(jnp.int32, L)

    @plsc.parallel_loop(0, b_per_w // L)
    def sum_body(blk):
      x = jnp.zeros((L,), jnp.float32)
      for j in range(L):
        pv = part[pl.ds((blk * L + j) * L, L)]
        x = jnp.where(lanes == j, jnp.sum(pv), x)
      i = plsc.bitcast(x, jnp.int32)
      i = jnp.int32(0x5F3759DF) - (i >> 1)
      y = plsc.bitcast(i, jnp.float32)
      half_x = 0.5 * x
      for _ in range(3):
        y = y * (1.5 - half_x * y * y)
      obuf[pl.ds(blk * L, L)] = -(x * y)

    pltpu.sync_copy(obuf, out_hbm.at[pl.ds(base_row, b_per_w)])

  return k


def kernel(head, relation, tail, entity_emb, relation_emb):
  batch = head.shape[0]
  return _tsvec_sc(batch)(head, relation, tail, entity_emb, relation_emb)


# speedup vs baseline: 1.3457x; 1.0540x over previous
import functools

import jax
import jax.numpy as jnp
from jax import lax
from jax.experimental import pallas as pl
from jax.experimental.pallas import tpu as pltpu
from jax.experimental.pallas import tpu_sc as plsc

D = 128
L = 16
NW = 32
CH = 128
NSLOT = 3


def _tsvec_sc(batch):
  b_per_w = batch // NW
  n_chunks = b_per_w // CH
  mesh = plsc.VectorSubcoreMesh(core_axis_name="c", subcore_axis_name="s")

  @functools.partial(
      pl.kernel,
      mesh=mesh,
      compiler_params=pltpu.CompilerParams(needs_layout_passes=False),
      out_type=jax.ShapeDtypeStruct((batch,), jnp.float32),
      scratch_types=[
          pltpu.VMEM((n_chunks, CH), jnp.int32),
          pltpu.VMEM((n_chunks, CH), jnp.int32),
          pltpu.VMEM((n_chunks, CH), jnp.int32),
          pltpu.VMEM((NSLOT, CH, D), jnp.float32),
          pltpu.VMEM((NSLOT, CH, D), jnp.float32),
          pltpu.VMEM((b_per_w * L,), jnp.float32),
          pltpu.VMEM((b_per_w,), jnp.float32),
          pltpu.SemaphoreType.DMA((NSLOT,)),
          pltpu.SemaphoreType.DMA((NSLOT,)),
          pltpu.SemaphoreType.DMA((NSLOT,)),
          pltpu.SemaphoreType.DMA,
          pltpu.VMEM_SHARED((1000, D), jnp.float32),
      ],
  )
  def k(head_hbm, rel_hbm, tail_hbm, ent_hbm, relemb_hbm, out_hbm,
        hidx, ridx, tidx, hbuf, tbuf, part, obuf, hsem, tsem, rsem, isem, rel_sp):
    wid = lax.axis_index("s") * 2 + lax.axis_index("c")
    base_row = wid * b_per_w

    @pl.when(lax.axis_index("s") == 0)
    def _():
      pltpu.async_copy(relemb_hbm, rel_sp, isem).wait()

    plsc.subcore_barrier()
    idx_copies = []
    for ci in range(n_chunks):
      span = pl.ds(base_row + ci * CH, CH)
      idx_copies.append(pltpu.async_copy(head_hbm.at[span], hidx.at[ci], isem))
      idx_copies.append(pltpu.async_copy(rel_hbm.at[span], ridx.at[ci], isem))
      idx_copies.append(pltpu.async_copy(tail_hbm.at[span], tidx.at[ci], isem))
    for c in idx_copies:
      c.wait()

    def fire_ht(ci):
      slot = ci % NSLOT
      pltpu.async_copy(ent_hbm.at[hidx.at[ci]], hbuf.at[slot], hsem.at[slot])
      pltpu.async_copy(ent_hbm.at[tidx.at[ci]], tbuf.at[slot], tsem.at[slot])

    def wait_h(ci):
      slot = ci % NSLOT
      pltpu.make_async_copy(ent_hbm.at[hidx.at[ci]], hbuf.at[slot],
                            hsem.at[slot]).wait()

    def fire_radd(ci):
      slot = ci % NSLOT
      pltpu.async_copy(rel_sp.at[ridx.at[ci]], hbuf.at[slot],
                       rsem.at[slot], add=True)

    def wait_tr(ci):
      slot = ci % NSLOT
      pltpu.make_async_copy(ent_hbm.at[tidx.at[ci]], tbuf.at[slot],
                            tsem.at[slot]).wait()
      pltpu.make_async_copy(rel_sp.at[ridx.at[ci]], hbuf.at[slot],
                            rsem.at[slot]).wait()

    fire_ht(0)
    if n_chunks > 1:
      fire_ht(1)
    wait_h(0)
    fire_radd(0)

    for ci in range(n_chunks):
      slot = ci % NSLOT
      if ci + 2 < n_chunks:
        fire_ht(ci + 2)
      if ci + 1 < n_chunks:
        wait_h(ci + 1)
        fire_radd(ci + 1)
      wait_tr(ci)

      U = 1

      @plsc.parallel_loop(0, CH // U)
      def row_body(it):
        base = it * U
        for j in range(U):
          acc = jnp.zeros((L,), jnp.float32)
          for c in range(D // L):
            hv = hbuf[slot, base + j, pl.ds(c * L, L)]
            tv = tbuf[slot, base + j, pl.ds(c * L, L)]
            d = hv - tv
            acc = acc + d * d
          part[pl.ds((ci * CH + base + j) * L, L)] = acc

    lanes = lax.iota(jnp.int32, L)

    @plsc.parallel_loop(0, b_per_w // L)
    def sum_body(blk):
      x = jnp.zeros((L,), jnp.float32)
      for j in range(L):
        pv = part[pl.ds((blk * L + j) * L, L)]
        x = jnp.where(lanes == j, jnp.sum(pv), x)
      i = plsc.bitcast(x, jnp.int32)
      i = jnp.int32(0x5F3759DF) - (i >> 1)
      y = plsc.bitcast(i, jnp.float32)
      half_x = 0.5 * x
      for _ in range(3):
        y = y * (1.5 - half_x * y * y)
      obuf[pl.ds(blk * L, L)] = -(x * y)

    pltpu.sync_copy(obuf, out_hbm.at[pl.ds(base_row, b_per_w)])

  return k


def kernel(head, relation, tail, entity_emb, relation_emb):
  batch = head.shape[0]
  return _tsvec_sc(batch)(head, relation, tail, entity_emb, relation_emb)


# speedup vs baseline: 1.3873x; 1.0309x over previous
import functools

import jax
import jax.numpy as jnp
from jax import lax
from jax.experimental import pallas as pl
from jax.experimental.pallas import tpu as pltpu
from jax.experimental.pallas import tpu_sc as plsc

D = 128
L = 16
NW = 32
CH = 128
NSLOT = 3


def _tsvec_sc(batch):
  b_per_w = batch // NW
  n_chunks = b_per_w // CH
  mesh = plsc.VectorSubcoreMesh(core_axis_name="c", subcore_axis_name="s")

  @functools.partial(
      pl.kernel,
      mesh=mesh,
      compiler_params=pltpu.CompilerParams(needs_layout_passes=False),
      out_type=jax.ShapeDtypeStruct((batch,), jnp.float32),
      scratch_types=[
          pltpu.VMEM((n_chunks, CH), jnp.int32),
          pltpu.VMEM((n_chunks, CH), jnp.int32),
          pltpu.VMEM((n_chunks, CH), jnp.int32),
          pltpu.VMEM((NSLOT, CH, D), jnp.float32),
          pltpu.VMEM((NSLOT, CH, D), jnp.float32),
          pltpu.VMEM((b_per_w * L,), jnp.float32),
          pltpu.VMEM((b_per_w,), jnp.float32),
          pltpu.SemaphoreType.DMA((NSLOT,)),
          pltpu.SemaphoreType.DMA((NSLOT,)),
          pltpu.SemaphoreType.DMA((NSLOT,)),
          pltpu.SemaphoreType.DMA,
          pltpu.VMEM_SHARED((1000, D), jnp.float32),
      ],
  )
  def k(head_hbm, rel_hbm, tail_hbm, ent_hbm, relemb_hbm, out_hbm,
        hidx, ridx, tidx, hbuf, tbuf, part, obuf, hsem, tsem, rsem, isem, rel_sp):
    wid = lax.axis_index("s") * 2 + lax.axis_index("c")
    base_row = wid * b_per_w

    sid0 = lax.axis_index("s") == 0

    @pl.when(sid0)
    def _():
      pltpu.async_copy(relemb_hbm, rel_sp, isem)

    idx_copies = []
    for ci in range(n_chunks):
      span = pl.ds(base_row + ci * CH, CH)
      idx_copies.append(pltpu.async_copy(head_hbm.at[span], hidx.at[ci], isem))
      idx_copies.append(pltpu.async_copy(rel_hbm.at[span], ridx.at[ci], isem))
      idx_copies.append(pltpu.async_copy(tail_hbm.at[span], tidx.at[ci], isem))
    for c in idx_copies:
      c.wait()

    def fire_ht(ci):
      slot = ci % NSLOT
      pltpu.async_copy(ent_hbm.at[hidx.at[ci]], hbuf.at[slot], hsem.at[slot])
      pltpu.async_copy(ent_hbm.at[tidx.at[ci]], tbuf.at[slot], tsem.at[slot])

    def wait_h(ci):
      slot = ci % NSLOT
      pltpu.make_async_copy(ent_hbm.at[hidx.at[ci]], hbuf.at[slot],
                            hsem.at[slot]).wait()

    def fire_radd(ci):
      slot = ci % NSLOT
      pltpu.async_copy(rel_sp.at[ridx.at[ci]], hbuf.at[slot],
                       rsem.at[slot], add=True)

    def wait_tr(ci):
      slot = ci % NSLOT
      pltpu.make_async_copy(ent_hbm.at[tidx.at[ci]], tbuf.at[slot],
                            tsem.at[slot]).wait()
      pltpu.make_async_copy(rel_sp.at[ridx.at[ci]], hbuf.at[slot],
                            rsem.at[slot]).wait()

    fire_ht(0)
    if n_chunks > 1:
      fire_ht(1)

    @pl.when(sid0)
    def _():
      pltpu.make_async_copy(relemb_hbm, rel_sp, isem).wait()

    plsc.subcore_barrier()
    wait_h(0)
    fire_radd(0)

    for ci in range(n_chunks):
      slot = ci % NSLOT
      if ci + 2 < n_chunks:
        fire_ht(ci + 2)
      if ci + 1 < n_chunks:
        wait_h(ci + 1)
        fire_radd(ci + 1)
      wait_tr(ci)

      U = 1

      @plsc.parallel_loop(0, CH // U)
      def row_body(it):
        base = it * U
        for j in range(U):
          acc = jnp.zeros((L,), jnp.float32)
          for c in range(D // L):
            hv = hbuf[slot, base + j, pl.ds(c * L, L)]
            tv = tbuf[slot, base + j, pl.ds(c * L, L)]
            d = hv - tv
            acc = acc + d * d
          part[pl.ds((ci * CH + base + j) * L, L)] = acc

    lanes = lax.iota(jnp.int32, L)

    @plsc.parallel_loop(0, b_per_w // L)
    def sum_body(blk):
      x = jnp.zeros((L,), jnp.float32)
      for j in range(L):
        pv = part[pl.ds((blk * L + j) * L, L)]
        x = jnp.where(lanes == j, jnp.sum(pv), x)
      i = plsc.bitcast(x, jnp.int32)
      i = jnp.int32(0x5F3759DF) - (i >> 1)
      y = plsc.bitcast(i, jnp.float32)
      half_x = 0.5 * x
      for _ in range(3):
        y = y * (1.5 - half_x * y * y)
      obuf[pl.ds(blk * L, L)] = -(x * y)

    pltpu.sync_copy(obuf, out_hbm.at[pl.ds(base_row, b_per_w)])

  return k


def kernel(head, relation, tail, entity_emb, relation_emb):
  batch = head.shape[0]
  return _tsvec_sc(batch)(head, relation, tail, entity_emb, relation_emb)


# speedup vs baseline: 1.3934x; 1.0044x over previous
import functools

import jax
import jax.numpy as jnp
from jax import lax
from jax.experimental import pallas as pl
from jax.experimental.pallas import tpu as pltpu
from jax.experimental.pallas import tpu_sc as plsc

D = 128
L = 16
NW = 32
CH = 64
NSLOT = 3


def _tsvec_sc(batch):
  b_per_w = batch // NW
  n_chunks = b_per_w // CH
  mesh = plsc.VectorSubcoreMesh(core_axis_name="c", subcore_axis_name="s")

  @functools.partial(
      pl.kernel,
      mesh=mesh,
      compiler_params=pltpu.CompilerParams(needs_layout_passes=False),
      out_type=jax.ShapeDtypeStruct((batch,), jnp.float32),
      scratch_types=[
          pltpu.VMEM((n_chunks, CH), jnp.int32),
          pltpu.VMEM((n_chunks, CH), jnp.int32),
          pltpu.VMEM((n_chunks, CH), jnp.int32),
          pltpu.VMEM((NSLOT, CH, D), jnp.float32),
          pltpu.VMEM((NSLOT, CH, D), jnp.float32),
          pltpu.VMEM((b_per_w * L,), jnp.float32),
          pltpu.VMEM((b_per_w,), jnp.float32),
          pltpu.SemaphoreType.DMA((NSLOT,)),
          pltpu.SemaphoreType.DMA((NSLOT,)),
          pltpu.SemaphoreType.DMA((NSLOT,)),
          pltpu.SemaphoreType.DMA,
          pltpu.VMEM_SHARED((1000, D), jnp.float32),
      ],
  )
  def k(head_hbm, rel_hbm, tail_hbm, ent_hbm, relemb_hbm, out_hbm,
        hidx, ridx, tidx, hbuf, tbuf, part, obuf, hsem, tsem, rsem, isem, rel_sp):
    wid = lax.axis_index("s") * 2 + lax.axis_index("c")
    base_row = wid * b_per_w

    sid0 = lax.axis_index("s") == 0

    @pl.when(sid0)
    def _():
      pltpu.async_copy(relemb_hbm, rel_sp, isem)

    idx_copies = []
    for ci in range(n_chunks):
      span = pl.ds(base_row + ci * CH, CH)
      idx_copies.append(pltpu.async_copy(head_hbm.at[span], hidx.at[ci], isem))
      idx_copies.append(pltpu.async_copy(rel_hbm.at[span], ridx.at[ci], isem))
      idx_copies.append(pltpu.async_copy(tail_hbm.at[span], tidx.at[ci], isem))
    for c in idx_copies:
      c.wait()

    def fire_ht(ci):
      slot = ci % NSLOT
      pltpu.async_copy(ent_hbm.at[hidx.at[ci]], hbuf.at[slot], hsem.at[slot])
      pltpu.async_copy(ent_hbm.at[tidx.at[ci]], tbuf.at[slot], tsem.at[slot])

    def wait_h(ci):
      slot = ci % NSLOT
      pltpu.make_async_copy(ent_hbm.at[hidx.at[ci]], hbuf.at[slot],
                            hsem.at[slot]).wait()

    def fire_radd(ci):
      slot = ci % NSLOT
      pltpu.async_copy(rel_sp.at[ridx.at[ci]], hbuf.at[slot],
                       rsem.at[slot], add=True)

    def wait_tr(ci):
      slot = ci % NSLOT
      pltpu.make_async_copy(ent_hbm.at[tidx.at[ci]], tbuf.at[slot],
                            tsem.at[slot]).wait()
      pltpu.make_async_copy(rel_sp.at[ridx.at[ci]], hbuf.at[slot],
                            rsem.at[slot]).wait()

    fire_ht(0)
    if n_chunks > 1:
      fire_ht(1)

    @pl.when(sid0)
    def _():
      pltpu.make_async_copy(relemb_hbm, rel_sp, isem).wait()

    plsc.subcore_barrier()
    wait_h(0)
    fire_radd(0)

    for ci in range(n_chunks):
      slot = ci % NSLOT
      if ci + 2 < n_chunks:
        fire_ht(ci + 2)
      if ci + 1 < n_chunks:
        wait_h(ci + 1)
        fire_radd(ci + 1)
      wait_tr(ci)

      U = 1

      @plsc.parallel_loop(0, CH // U)
      def row_body(it):
        base = it * U
        for j in range(U):
          acc = jnp.zeros((L,), jnp.float32)
          for c in range(D // L):
            hv = hbuf[slot, base + j, pl.ds(c * L, L)]
            tv = tbuf[slot, base + j, pl.ds(c * L, L)]
            d = hv - tv
            acc = acc + d * d
          part[pl.ds((ci * CH + base + j) * L, L)] = acc

    lanes = lax.iota(jnp.int32, L)

    @plsc.parallel_loop(0, b_per_w // L)
    def sum_body(blk):
      x = jnp.zeros((L,), jnp.float32)
      for j in range(L):
        pv = part[pl.ds((blk * L + j) * L, L)]
        x = jnp.where(lanes == j, jnp.sum(pv), x)
      i = plsc.bitcast(x, jnp.int32)
      i = jnp.int32(0x5F3759DF) - (i >> 1)
      y = plsc.bitcast(i, jnp.float32)
      half_x = 0.5 * x
      for _ in range(3):
        y = y * (1.5 - half_x * y * y)
      obuf[pl.ds(blk * L, L)] = -(x * y)

    pltpu.sync_copy(obuf, out_hbm.at[pl.ds(base_row, b_per_w)])

  return k


def kernel(head, relation, tail, entity_emb, relation_emb):
  batch = head.shape[0]
  return _tsvec_sc(batch)(head, relation, tail, entity_emb, relation_emb)


# speedup vs baseline: 1.4172x; 1.0171x over previous
import functools

import jax
import jax.numpy as jnp
from jax import lax
from jax.experimental import pallas as pl
from jax.experimental.pallas import tpu as pltpu
from jax.experimental.pallas import tpu_sc as plsc

D = 128
L = 16
NW = 32
CH = 64
NSLOT = 4


def _tsvec_sc(batch):
  b_per_w = batch // NW
  n_chunks = b_per_w // CH
  mesh = plsc.VectorSubcoreMesh(core_axis_name="c", subcore_axis_name="s")

  @functools.partial(
      pl.kernel,
      mesh=mesh,
      compiler_params=pltpu.CompilerParams(needs_layout_passes=False),
      out_type=jax.ShapeDtypeStruct((batch,), jnp.float32),
      scratch_types=[
          pltpu.VMEM((n_chunks, CH), jnp.int32),
          pltpu.VMEM((n_chunks, CH), jnp.int32),
          pltpu.VMEM((n_chunks, CH), jnp.int32),
          pltpu.VMEM((NSLOT, CH, D), jnp.float32),
          pltpu.VMEM((NSLOT, CH, D), jnp.float32),
          pltpu.VMEM((b_per_w * L,), jnp.float32),
          pltpu.VMEM((b_per_w,), jnp.float32),
          pltpu.SemaphoreType.DMA((NSLOT,)),
          pltpu.SemaphoreType.DMA((NSLOT,)),
          pltpu.SemaphoreType.DMA((NSLOT,)),
          pltpu.SemaphoreType.DMA,
          pltpu.VMEM_SHARED((1000, D), jnp.float32),
      ],
  )
  def k(head_hbm, rel_hbm, tail_hbm, ent_hbm, relemb_hbm, out_hbm,
        hidx, ridx, tidx, hbuf, tbuf, part, obuf, hsem, tsem, rsem, isem, rel_sp):
    wid = lax.axis_index("s") * 2 + lax.axis_index("c")
    base_row = wid * b_per_w

    sid0 = lax.axis_index("s") == 0

    @pl.when(sid0)
    def _():
      pltpu.async_copy(relemb_hbm, rel_sp, isem)

    idx_copies = []
    for ci in range(n_chunks):
      span = pl.ds(base_row + ci * CH, CH)
      idx_copies.append(pltpu.async_copy(head_hbm.at[span], hidx.at[ci], isem))
      idx_copies.append(pltpu.async_copy(rel_hbm.at[span], ridx.at[ci], isem))
      idx_copies.append(pltpu.async_copy(tail_hbm.at[span], tidx.at[ci], isem))
    for c in idx_copies:
      c.wait()

    def fire_ht(ci):
      slot = ci % NSLOT
      pltpu.async_copy(ent_hbm.at[hidx.at[ci]], hbuf.at[slot], hsem.at[slot])
      pltpu.async_copy(ent_hbm.at[tidx.at[ci]], tbuf.at[slot], tsem.at[slot])

    def wait_h(ci):
      slot = ci % NSLOT
      pltpu.make_async_copy(ent_hbm.at[hidx.at[ci]], hbuf.at[slot],
                            hsem.at[slot]).wait()

    def fire_radd(ci):
      slot = ci % NSLOT
      pltpu.async_copy(rel_sp.at[ridx.at[ci]], hbuf.at[slot],
                       rsem.at[slot], add=True)

    def wait_tr(ci):
      slot = ci % NSLOT
      pltpu.make_async_copy(ent_hbm.at[tidx.at[ci]], tbuf.at[slot],
                            tsem.at[slot]).wait()
      pltpu.make_async_copy(rel_sp.at[ridx.at[ci]], hbuf.at[slot],
                            rsem.at[slot]).wait()

    fire_ht(0)
    if n_chunks > 1:
      fire_ht(1)
    if n_chunks > 2:
      fire_ht(2)

    @pl.when(sid0)
    def _():
      pltpu.make_async_copy(relemb_hbm, rel_sp, isem).wait()

    plsc.subcore_barrier()
    wait_h(0)
    fire_radd(0)

    for ci in range(n_chunks):
      slot = ci % NSLOT
      if ci + 3 < n_chunks:
        fire_ht(ci + 3)
      if ci + 1 < n_chunks:
        wait_h(ci + 1)
        fire_radd(ci + 1)
      wait_tr(ci)

      U = 1

      @plsc.parallel_loop(0, CH // U)
      def row_body(it):
        base = it * U
        for j in range(U):
          acc = jnp.zeros((L,), jnp.float32)
          for c in range(D // L):
            hv = hbuf[slot, base + j, pl.ds(c * L, L)]
            tv = tbuf[slot, base + j, pl.ds(c * L, L)]
            d = hv - tv
            acc = acc + d * d
          part[pl.ds((ci * CH + base + j) * L, L)] = acc

    lanes = lax.iota(jnp.int32, L)

    @plsc.parallel_loop(0, b_per_w // L)
    def sum_body(blk):
      x = jnp.zeros((L,), jnp.float32)
      for j in range(L):
        pv = part[pl.ds((blk * L + j) * L, L)]
        x = jnp.where(lanes == j, jnp.sum(pv), x)
      i = plsc.bitcast(x, jnp.int32)
      i = jnp.int32(0x5F3759DF) - (i >> 1)
      y = plsc.bitcast(i, jnp.float32)
      half_x = 0.5 * x
      for _ in range(3):
        y = y * (1.5 - half_x * y * y)
      obuf[pl.ds(blk * L, L)] = -(x * y)

    pltpu.sync_copy(obuf, out_hbm.at[pl.ds(base_row, b_per_w)])

  return k


def kernel(head, relation, tail, entity_emb, relation_emb):
  batch = head.shape[0]
  return _tsvec_sc(batch)(head, relation, tail, entity_emb, relation_emb)


# speedup vs baseline: 1.4326x; 1.0109x over previous
import functools

import jax
import jax.numpy as jnp
from jax import lax
from jax.experimental import pallas as pl
from jax.experimental.pallas import tpu as pltpu
from jax.experimental.pallas import tpu_sc as plsc

D = 128
L = 16
NW = 32
CH = 64
NSLOT = 4


def _tsvec_sc(batch):
  b_per_w = batch // NW
  n_chunks = b_per_w // CH
  mesh = plsc.VectorSubcoreMesh(core_axis_name="c", subcore_axis_name="s")

  @functools.partial(
      pl.kernel,
      mesh=mesh,
      compiler_params=pltpu.CompilerParams(needs_layout_passes=False),
      out_type=jax.ShapeDtypeStruct((batch,), jnp.float32),
      scratch_types=[
          pltpu.VMEM((b_per_w,), jnp.int32),
          pltpu.VMEM((b_per_w,), jnp.int32),
          pltpu.VMEM((b_per_w,), jnp.int32),
          pltpu.VMEM((NSLOT, CH, D), jnp.float32),
          pltpu.VMEM((NSLOT, CH, D), jnp.float32),
          pltpu.VMEM((b_per_w * L,), jnp.float32),
          pltpu.VMEM((b_per_w,), jnp.float32),
          pltpu.SemaphoreType.DMA((NSLOT,)),
          pltpu.SemaphoreType.DMA((NSLOT,)),
          pltpu.SemaphoreType.DMA((NSLOT,)),
          pltpu.SemaphoreType.DMA,
          pltpu.VMEM_SHARED((1000, D), jnp.float32),
      ],
  )
  def k(head_hbm, rel_hbm, tail_hbm, ent_hbm, relemb_hbm, out_hbm,
        hidx, ridx, tidx, hbuf, tbuf, part, obuf, hsem, tsem, rsem, isem, rel_sp):
    wid = lax.axis_index("s") * 2 + lax.axis_index("c")
    base_row = wid * b_per_w

    sid0 = lax.axis_index("s") == 0

    @pl.when(sid0)
    def _():
      pltpu.async_copy(relemb_hbm, rel_sp, isem)

    span = pl.ds(base_row, b_per_w)
    idx_copies = [
        pltpu.async_copy(head_hbm.at[span], hidx, isem),
        pltpu.async_copy(rel_hbm.at[span], ridx, isem),
        pltpu.async_copy(tail_hbm.at[span], tidx, isem),
    ]
    for c in idx_copies:
      c.wait()

    def fire_ht(ci):
      slot = ci % NSLOT
      pltpu.async_copy(ent_hbm.at[hidx.at[pl.ds(ci * CH, CH)]], hbuf.at[slot], hsem.at[slot])
      pltpu.async_copy(ent_hbm.at[tidx.at[pl.ds(ci * CH, CH)]], tbuf.at[slot], tsem.at[slot])

    def wait_h(ci):
      slot = ci % NSLOT
      pltpu.make_async_copy(ent_hbm.at[hidx.at[pl.ds(ci * CH, CH)]], hbuf.at[slot],
                            hsem.at[slot]).wait()

    def fire_radd(ci):
      slot = ci % NSLOT
      pltpu.async_copy(rel_sp.at[ridx.at[pl.ds(ci * CH, CH)]], hbuf.at[slot],
                       rsem.at[slot], add=True)

    def wait_tr(ci):
      slot = ci % NSLOT
      pltpu.make_async_copy(ent_hbm.at[tidx.at[pl.ds(ci * CH, CH)]], tbuf.at[slot],
                            tsem.at[slot]).wait()
      pltpu.make_async_copy(rel_sp.at[ridx.at[pl.ds(ci * CH, CH)]], hbuf.at[slot],
                            rsem.at[slot]).wait()

    fire_ht(0)
    if n_chunks > 1:
      fire_ht(1)
    if n_chunks > 2:
      fire_ht(2)

    @pl.when(sid0)
    def _():
      pltpu.make_async_copy(relemb_hbm, rel_sp, isem).wait()

    plsc.subcore_barrier()
    wait_h(0)
    fire_radd(0)

    for ci in range(n_chunks):
      slot = ci % NSLOT
      if ci + 3 < n_chunks:
        fire_ht(ci + 3)
      if ci + 1 < n_chunks:
        wait_h(ci + 1)
        fire_radd(ci + 1)
      wait_tr(ci)

      U = 1

      @plsc.parallel_loop(0, CH // U)
      def row_body(it):
        base = it * U
        for j in range(U):
          acc = jnp.zeros((L,), jnp.float32)
          for c in range(D // L):
            hv = hbuf[slot, base + j, pl.ds(c * L, L)]
            tv = tbuf[slot, base + j, pl.ds(c * L, L)]
            d = hv - tv
            acc = acc + d * d
          part[pl.ds((ci * CH + base + j) * L, L)] = acc

    lanes = lax.iota(jnp.int32, L)

    @plsc.parallel_loop(0, b_per_w // L)
    def sum_body(blk):
      x = jnp.zeros((L,), jnp.float32)
      for j in range(L):
        pv = part[pl.ds((blk * L + j) * L, L)]
        x = jnp.where(lanes == j, jnp.sum(pv), x)
      i = plsc.bitcast(x, jnp.int32)
      i = jnp.int32(0x5F3759DF) - (i >> 1)
      y = plsc.bitcast(i, jnp.float32)
      half_x = 0.5 * x
      for _ in range(3):
        y = y * (1.5 - half_x * y * y)
      obuf[pl.ds(blk * L, L)] = -(x * y)

    pltpu.sync_copy(obuf, out_hbm.at[pl.ds(base_row, b_per_w)])

  return k


def kernel(head, relation, tail, entity_emb, relation_emb):
  batch = head.shape[0]
  return _tsvec_sc(batch)(head, relation, tail, entity_emb, relation_emb)


# speedup vs baseline: 1.4909x; 1.0407x over previous
import functools

import jax
import jax.numpy as jnp
from jax import lax
from jax.experimental import pallas as pl
from jax.experimental.pallas import tpu as pltpu
from jax.experimental.pallas import tpu_sc as plsc

D = 128
L = 16
NW = 32
CH = 64
NSLOT = 4


def _tsvec_sc(batch):
  b_per_w = batch // NW
  n_chunks = b_per_w // CH
  mesh = plsc.VectorSubcoreMesh(core_axis_name="c", subcore_axis_name="s")

  @functools.partial(
      pl.kernel,
      mesh=mesh,
      compiler_params=pltpu.CompilerParams(needs_layout_passes=False),
      out_type=jax.ShapeDtypeStruct((batch,), jnp.float32),
      scratch_types=[
          pltpu.VMEM((b_per_w,), jnp.int32),
          pltpu.VMEM((b_per_w,), jnp.int32),
          pltpu.VMEM((b_per_w,), jnp.int32),
          pltpu.VMEM((NSLOT, CH, D), jnp.float32),
          pltpu.VMEM((NSLOT, CH, D), jnp.float32),
          pltpu.VMEM((b_per_w * L,), jnp.float32),
          pltpu.VMEM((b_per_w,), jnp.float32),
          pltpu.SemaphoreType.DMA((NSLOT,)),
          pltpu.SemaphoreType.DMA((NSLOT,)),
          pltpu.SemaphoreType.DMA((NSLOT,)),
          pltpu.SemaphoreType.DMA,
          pltpu.VMEM_SHARED((1000, D), jnp.float32),
      ],
  )
  def k(head_hbm, rel_hbm, tail_hbm, ent_hbm, relemb_hbm, out_hbm,
        hidx, ridx, tidx, hbuf, tbuf, part, obuf, hsem, tsem, rsem, isem, rel_sp):
    wid = lax.axis_index("s") * 2 + lax.axis_index("c")
    base_row = wid * b_per_w

    sid0 = lax.axis_index("s") == 0

    @pl.when(sid0)
    def _():
      pltpu.async_copy(relemb_hbm, rel_sp, isem)

    span = pl.ds(base_row, b_per_w)
    idx_copies = [
        pltpu.async_copy(head_hbm.at[span], hidx, isem),
        pltpu.async_copy(rel_hbm.at[span], ridx, isem),
        pltpu.async_copy(tail_hbm.at[span], tidx, isem),
    ]
    for c in idx_copies:
      c.wait()

    def fire_ht(ci):
      slot = lax.rem(ci, NSLOT) if not isinstance(ci, int) else ci % NSLOT
      pltpu.async_copy(ent_hbm.at[hidx.at[pl.ds(ci * CH, CH)]], hbuf.at[slot], hsem.at[slot])
      pltpu.async_copy(ent_hbm.at[tidx.at[pl.ds(ci * CH, CH)]], tbuf.at[slot], tsem.at[slot])

    def wait_h(ci):
      slot = lax.rem(ci, NSLOT) if not isinstance(ci, int) else ci % NSLOT
      pltpu.make_async_copy(ent_hbm.at[hidx.at[pl.ds(ci * CH, CH)]], hbuf.at[slot],
                            hsem.at[slot]).wait()

    def fire_radd(ci):
      slot = lax.rem(ci, NSLOT) if not isinstance(ci, int) else ci % NSLOT
      pltpu.async_copy(rel_sp.at[ridx.at[pl.ds(ci * CH, CH)]], hbuf.at[slot],
                       rsem.at[slot], add=True)

    def wait_tr(ci):
      slot = lax.rem(ci, NSLOT) if not isinstance(ci, int) else ci % NSLOT
      pltpu.make_async_copy(ent_hbm.at[tidx.at[pl.ds(ci * CH, CH)]], tbuf.at[slot],
                            tsem.at[slot]).wait()
      pltpu.make_async_copy(rel_sp.at[ridx.at[pl.ds(ci * CH, CH)]], hbuf.at[slot],
                            rsem.at[slot]).wait()

    fire_ht(0)
    if n_chunks > 1:
      fire_ht(1)
    if n_chunks > 2:
      fire_ht(2)

    @pl.when(sid0)
    def _():
      pltpu.make_async_copy(relemb_hbm, rel_sp, isem).wait()

    plsc.subcore_barrier()
    wait_h(0)
    fire_radd(0)

    def chunk_body(ci, _):
      slot = lax.rem(ci, NSLOT)

      @pl.when(ci + 3 < n_chunks)
      def _():
        fire_ht(ci + 3)

      @pl.when(ci + 1 < n_chunks)
      def _():
        wait_h(ci + 1)
        fire_radd(ci + 1)

      wait_tr(ci)

      @plsc.parallel_loop(0, CH)
      def row_body(it):
        acc = jnp.zeros((L,), jnp.float32)
        for c in range(D // L):
          hv = hbuf[slot, it, pl.ds(c * L, L)]
          tv = tbuf[slot, it, pl.ds(c * L, L)]
          d = hv - tv
          acc = acc + d * d
        part[pl.ds((ci * CH + it) * L, L)] = acc

      return 0

    lax.fori_loop(0, n_chunks, chunk_body, 0)

    lanes = lax.iota(jnp.int32, L)

    @plsc.parallel_loop(0, b_per_w // L)
    def sum_body(blk):
      x = jnp.zeros((L,), jnp.float32)
      for j in range(L):
        pv = part[pl.ds((blk * L + j) * L, L)]
        x = jnp.where(lanes == j, jnp.sum(pv), x)
      i = plsc.bitcast(x, jnp.int32)
      i = jnp.int32(0x5F3759DF) - (i >> 1)
      y = plsc.bitcast(i, jnp.float32)
      half_x = 0.5 * x
      for _ in range(3):
        y = y * (1.5 - half_x * y * y)
      obuf[pl.ds(blk * L, L)] = -(x * y)

    pltpu.sync_copy(obuf, out_hbm.at[pl.ds(base_row, b_per_w)])

  return k


def kernel(head, relation, tail, entity_emb, relation_emb):
  batch = head.shape[0]
  return _tsvec_sc(batch)(head, relation, tail, entity_emb, relation_emb)


# speedup vs baseline: 1.4956x; 1.0032x over previous
import functools

import jax
import jax.numpy as jnp
from jax import lax
from jax.experimental import pallas as pl
from jax.experimental.pallas import tpu as pltpu
from jax.experimental.pallas import tpu_sc as plsc

D = 128
L = 16
NW = 32
CH = 64
NSLOT = 5


def _tsvec_sc(batch):
  b_per_w = batch // NW
  n_chunks = b_per_w // CH
  mesh = plsc.VectorSubcoreMesh(core_axis_name="c", subcore_axis_name="s")

  @functools.partial(
      pl.kernel,
      mesh=mesh,
      compiler_params=pltpu.CompilerParams(needs_layout_passes=False),
      out_type=jax.ShapeDtypeStruct((batch,), jnp.float32),
      scratch_types=[
          pltpu.VMEM((b_per_w,), jnp.int32),
          pltpu.VMEM((b_per_w,), jnp.int32),
          pltpu.VMEM((b_per_w,), jnp.int32),
          pltpu.VMEM((NSLOT, CH, D), jnp.float32),
          pltpu.VMEM((NSLOT, CH, D), jnp.float32),
          pltpu.VMEM((b_per_w * L,), jnp.float32),
          pltpu.VMEM((b_per_w,), jnp.float32),
          pltpu.SemaphoreType.DMA((NSLOT,)),
          pltpu.SemaphoreType.DMA((NSLOT,)),
          pltpu.SemaphoreType.DMA((NSLOT,)),
          pltpu.SemaphoreType.DMA,
          pltpu.VMEM_SHARED((1000, D), jnp.float32),
      ],
  )
  def k(head_hbm, rel_hbm, tail_hbm, ent_hbm, relemb_hbm, out_hbm,
        hidx, ridx, tidx, hbuf, tbuf, part, obuf, hsem, tsem, rsem, isem, rel_sp):
    wid = lax.axis_index("s") * 2 + lax.axis_index("c")
    base_row = wid * b_per_w

    sid0 = lax.axis_index("s") == 0

    @pl.when(sid0)
    def _():
      pltpu.async_copy(relemb_hbm, rel_sp, isem)

    span = pl.ds(base_row, b_per_w)
    idx_copies = [
        pltpu.async_copy(head_hbm.at[span], hidx, isem),
        pltpu.async_copy(rel_hbm.at[span], ridx, isem),
        pltpu.async_copy(tail_hbm.at[span], tidx, isem),
    ]
    for c in idx_copies:
      c.wait()

    def fire_ht(ci):
      slot = lax.rem(ci, NSLOT) if not isinstance(ci, int) else ci % NSLOT
      pltpu.async_copy(ent_hbm.at[hidx.at[pl.ds(ci * CH, CH)]], hbuf.at[slot], hsem.at[slot])
      pltpu.async_copy(ent_hbm.at[tidx.at[pl.ds(ci * CH, CH)]], tbuf.at[slot], tsem.at[slot])

    def wait_h(ci):
      slot = lax.rem(ci, NSLOT) if not isinstance(ci, int) else ci % NSLOT
      pltpu.make_async_copy(ent_hbm.at[hidx.at[pl.ds(ci * CH, CH)]], hbuf.at[slot],
                            hsem.at[slot]).wait()

    def fire_radd(ci):
      slot = lax.rem(ci, NSLOT) if not isinstance(ci, int) else ci % NSLOT
      pltpu.async_copy(rel_sp.at[ridx.at[pl.ds(ci * CH, CH)]], hbuf.at[slot],
                       rsem.at[slot], add=True)

    def wait_tr(ci):
      slot = lax.rem(ci, NSLOT) if not isinstance(ci, int) else ci % NSLOT
      pltpu.make_async_copy(ent_hbm.at[tidx.at[pl.ds(ci * CH, CH)]], tbuf.at[slot],
                            tsem.at[slot]).wait()
      pltpu.make_async_copy(rel_sp.at[ridx.at[pl.ds(ci * CH, CH)]], hbuf.at[slot],
                            rsem.at[slot]).wait()

    fire_ht(0)
    if n_chunks > 1:
      fire_ht(1)
    if n_chunks > 2:
      fire_ht(2)
    if n_chunks > 3:
      fire_ht(3)

    @pl.when(sid0)
    def _():
      pltpu.make_async_copy(relemb_hbm, rel_sp, isem).wait()

    plsc.subcore_barrier()
    wait_h(0)
    fire_radd(0)

    def chunk_body(ci, _):
      slot = lax.rem(ci, NSLOT)

      @pl.when(ci + 4 < n_chunks)
      def _():
        fire_ht(ci + 4)

      @pl.when(ci + 1 < n_chunks)
      def _():
        wait_h(ci + 1)
        fire_radd(ci + 1)

      wait_tr(ci)

      @plsc.parallel_loop(0, CH)
      def row_body(it):
        acc = jnp.zeros((L,), jnp.float32)
        for c in range(D // L):
          hv = hbuf[slot, it, pl.ds(c * L, L)]
          tv = tbuf[slot, it, pl.ds(c * L, L)]
          d = hv - tv
          acc = acc + d * d
        part[pl.ds((ci * CH + it) * L, L)] = acc

      return 0

    lax.fori_loop(0, n_chunks, chunk_body, 0)

    lanes = lax.iota(jnp.int32, L)

    @plsc.parallel_loop(0, b_per_w // L)
    def sum_body(blk):
      x = jnp.zeros((L,), jnp.float32)
      for j in range(L):
        pv = part[pl.ds((blk * L + j) * L, L)]
        x = jnp.where(lanes == j, jnp.sum(pv), x)
      i = plsc.bitcast(x, jnp.int32)
      i = jnp.int32(0x5F3759DF) - (i >> 1)
      y = plsc.bitcast(i, jnp.float32)
      half_x = 0.5 * x
      for _ in range(3):
        y = y * (1.5 - half_x * y * y)
      obuf[pl.ds(blk * L, L)] = -(x * y)

    pltpu.sync_copy(obuf, out_hbm.at[pl.ds(base_row, b_per_w)])

  return k


def kernel(head, relation, tail, entity_emb, relation_emb):
  batch = head.shape[0]
  return _tsvec_sc(batch)(head, relation, tail, entity_emb, relation_emb)
